# baseline (speedup 1.0000x reference)
import functools

import jax
import jax.numpy as jnp
from jax import lax
from jax.experimental import pallas as pl
from jax.experimental.pallas import tpu as pltpu
from jax.experimental.pallas import tpu_sc as plsc

VOCAB = 1000000
PRETRAIN_DIM = 64
EMBED_DIM = 32
OOV_IDX = 999997


_SUB = 2048
_PACK = 4
_BLK = _SUB * _PACK
_GRID = -(-VOCAB // _BLK)
_DROWS = _GRID * _SUB
_VPAD = _DROWS * _PACK


_GDIM = PRETRAIN_DIM + EMBED_DIM
_OUTW = _PACK * EMBED_DIM


def _combine_body(pt_ref, it_ref, wf_ref, o_ref):
    i = pl.program_id(0)

    def slot_dots(j):
        proj = lax.dot_general(
            pt_ref[:, j * _SUB:(j + 1) * _SUB],
            wf_ref[j * _GDIM:j * _GDIM + PRETRAIN_DIM, :],
            dimension_numbers=(((0,), (0,)), ((), ())),
            preferred_element_type=jnp.float32,
        )
        idt = lax.dot_general(
            it_ref[:, j * _SUB:(j + 1) * _SUB],
            wf_ref[j * _GDIM + PRETRAIN_DIM:(j + 1) * _GDIM, :],
            dimension_numbers=(((0,), (0,)), ((), ())),
            preferred_element_type=jnp.float32,
        )
        return proj + idt

    @pl.when(i < _GRID - 1)
    def _():
        acc = None
        for j in range(_PACK):
            s = slot_dots(j)
            acc = s if acc is None else acc + s
        o_ref[...] = acc

    @pl.when(i == _GRID - 1)
    def _():
        s0 = slot_dots(0)
        col = lax.broadcasted_iota(jnp.int32, (_SUB, _OUTW), 1)
        v0 = i * _PACK * _SUB + lax.broadcasted_iota(
            jnp.int32, (_SUB, _OUTW), 0)
        keep = (col < EMBED_DIM) & (v0 <= OOV_IDX)
        o_ref[...] = jnp.where(keep, s0, 0.0)


def _combine(pretrain_table, id_table, W_proj):
    w96 = jnp.concatenate(
        [W_proj.T, jnp.eye(EMBED_DIM, dtype=jnp.float32)], axis=0)
    wfull = jnp.kron(jnp.eye(_PACK, dtype=jnp.float32), w96)
    packed = pl.pallas_call(
        _combine_body,
        grid=(_GRID,),
        in_specs=[
            pl.BlockSpec((PRETRAIN_DIM, _BLK), lambda i: (0, i)),
            pl.BlockSpec((EMBED_DIM, _BLK), lambda i: (0, i)),
            pl.BlockSpec((_PACK * _GDIM, _OUTW), lambda i: (0, 0)),
        ],
        out_specs=pl.BlockSpec((_SUB, _OUTW), lambda i: (i, 0)),
        out_shape=jax.ShapeDtypeStruct((_DROWS, _OUTW), jnp.float32),
    )(pretrain_table.T, id_table.T, wfull)
    return packed.reshape(_VPAD, EMBED_DIM)


_IDXW = 128
_STREAMS = 4
_CHUNK = _IDXW * _STREAMS


_UNROLL = 16


def _make_gather(b, l):
    n_tokens = b * l
    info = plsc.get_sparse_core_info()
    nc, ns = info.num_cores, info.num_subcores
    nw = nc * ns
    b_per_w = n_tokens // nw
    n_outer = b_per_w // _CHUNK
    chunks_per_l = b // _CHUNK
    mesh = plsc.VectorSubcoreMesh(core_axis_name="c", subcore_axis_name="s")

    @functools.partial(
        pl.kernel,
        mesh=mesh,
        out_type=jax.ShapeDtypeStruct((l, EMBED_DIM, b), jnp.float32),
        scratch_types=[
            pltpu.VMEM((_STREAMS, _IDXW), jnp.int32),
            pltpu.VMEM((_STREAMS, _IDXW), jnp.int32),
            pltpu.VMEM((_CHUNK, EMBED_DIM), jnp.float32),
            pltpu.VMEM((_CHUNK, EMBED_DIM), jnp.float32),
            pltpu.VMEM((EMBED_DIM * _CHUNK,), jnp.float32),
            pltpu.VMEM((EMBED_DIM * _CHUNK,), jnp.float32),
            pltpu.SemaphoreType.DMA,
            pltpu.SemaphoreType.DMA,
            pltpu.SemaphoreType.DMA,
            pltpu.SemaphoreType.DMA,
            pltpu.SemaphoreType.DMA,
            pltpu.SemaphoreType.DMA,
        ],
        compiler_params=pltpu.CompilerParams(
            use_tc_tiling_on_sc=False, needs_layout_passes=False),
    )
    def gather_k(table_hbm, idx_hbm, out_hbm, idx0, idx1, rows0, rows1,
                 t0, t1, si0, si1, sg0, sg1, so0, so1):
        idx = (idx0, idx1)
        rows = (rows0, rows1)
        t = (t0, t1)
        si = (si0, si1)
        sg = (sg0, sg1)
        so = (so0, so1)
        wid = lax.axis_index("s") * nc + lax.axis_index("c")
        irow_base = wid * (b_per_w // _IDXW)
        lane = lax.iota(jnp.int32, 16)
        col0 = lane * _CHUNK
        col1 = col0 + 16 * _CHUNK
        n = n_outer

        def idx_slice(i):
            return idx_hbm.at[
                pl.ds(pl.multiple_of(irow_base + i * _STREAMS, _STREAMS),
                      _STREAMS)]

        def fire_idx(i, p):
            pltpu.async_copy(idx_slice(i), idx[p], si[p])

        def wait_idx(i, p):
            pltpu.make_async_copy(idx_slice(i), idx[p], si[p]).wait()

        def fire_gathers(p):
            for j in range(_STREAMS):
                pltpu.async_copy(
                    table_hbm.at[idx[p].at[j]],
                    rows[p].at[pl.ds(j * _IDXW, _IDXW)],
                    sg[p],
                )

        def wait_gathers(p):
            pltpu.make_async_copy(
                table_hbm.at[pl.ds(0, _CHUNK)], rows[p], sg[p]).wait()

        def drain_out(p):
            pltpu.make_async_copy(out_hbm.at[0, 0], t[p], so[p]).wait()

        def chunk_pos(i):
            c = wid * n + i
            return c // chunks_per_l, (c % chunks_per_l) * _CHUNK

        def fire_out(i, p):
            li, b0 = chunk_pos(i)
            b0 = pl.multiple_of(b0, _CHUNK)
            for e in range(EMBED_DIM):
                pltpu.async_copy(
                    t[p].at[pl.ds(e * _CHUNK, _CHUNK)],
                    out_hbm.at[li, e, pl.ds(b0, _CHUNK)],
                    so[p],
                )

        def transpose(p):
            rv, tv = rows[p], t[p]

            @plsc.parallel_loop(0, _CHUNK, unroll=_UNROLL)
            def trow(rr):
                x0 = rv[rr, pl.ds(0, 16)]
                x1 = rv[rr, pl.ds(16, 16)]
                plsc.store_scatter(tv, [col0 + rr], x0)
                plsc.store_scatter(tv, [col1 + rr], x1)

        fire_idx(0, 0)
        fire_idx(1, 1)
        wait_idx(0, 0)
        fire_gathers(0)

        def step(g, _):
            for p in range(2):
                i = 2 * g + p
                q = 1 - p

                with jax.named_scope("fg"):
                    @pl.when(i + 1 < n)
                    def _():
                        wait_idx(i + 1, q)
                        fire_gathers(q)

                with jax.named_scope("gw"):
                    wait_gathers(p)

                    @pl.when(i + 2 < n)
                    def _():
                        fire_idx(i + 2, p)

                    @pl.when(i >= 2)
                    def _():
                        drain_out(p)

                with jax.named_scope("tr"):
                    transpose(p)
                with jax.named_scope("fo"):
                    fire_out(i, p)
            return _

        lax.fori_loop(0, n // 2, step, None)
        drain_out(0)
        drain_out(1)

    return gather_k


def kernel(inputs, pretrain_table, id_table, W_proj):
    b, l = inputs.shape
    n_tokens = b * l
    table = _combine(pretrain_table, id_table, W_proj)
    v = inputs.T.astype(jnp.int32)
    blk = v // _SUB
    k = ((blk // _PACK) * _SUB + v % _SUB) * _PACK + blk % _PACK
    idx2d = k.reshape(n_tokens // _IDXW, _IDXW)
    out_t = _make_gather(b, l)(table, idx2d)
    return out_t.transpose(2, 0, 1)


# speedup vs baseline: 1.6419x; 1.6419x over previous
import functools

import jax
import jax.numpy as jnp
from jax import lax
from jax.experimental import pallas as pl
from jax.experimental.pallas import tpu as pltpu
from jax.experimental.pallas import tpu_sc as plsc

VOCAB = 1000000
PRETRAIN_DIM = 64
EMBED_DIM = 32
OOV_IDX = 999997


_SUB = 2048
_PACK = 4
_BLK = _SUB * _PACK
_GRID = -(-VOCAB // _BLK)
_DROWS = _GRID * _SUB
_VPAD = _DROWS * _PACK


_GDIM = PRETRAIN_DIM + EMBED_DIM
_OUTW = _PACK * EMBED_DIM


def _combine_body(pt_ref, it_ref, wf_ref, o_ref):
    i = pl.program_id(0)

    def slot_dots(j):
        proj = lax.dot_general(
            pt_ref[:, j * _SUB:(j + 1) * _SUB],
            wf_ref[j * _GDIM:j * _GDIM + PRETRAIN_DIM, :],
            dimension_numbers=(((0,), (0,)), ((), ())),
            preferred_element_type=jnp.float32,
        )
        idt = lax.dot_general(
            it_ref[:, j * _SUB:(j + 1) * _SUB],
            wf_ref[j * _GDIM + PRETRAIN_DIM:(j + 1) * _GDIM, :],
            dimension_numbers=(((0,), (0,)), ((), ())),
            preferred_element_type=jnp.float32,
        )
        return proj + idt

    @pl.when(i < _GRID - 1)
    def _():
        acc = None
        for j in range(_PACK):
            s = slot_dots(j)
            acc = s if acc is None else acc + s
        o_ref[...] = acc

    @pl.when(i == _GRID - 1)
    def _():
        s0 = slot_dots(0)
        col = lax.broadcasted_iota(jnp.int32, (_SUB, _OUTW), 1)
        v0 = i * _PACK * _SUB + lax.broadcasted_iota(
            jnp.int32, (_SUB, _OUTW), 0)
        keep = (col < EMBED_DIM) & (v0 <= OOV_IDX)
        o_ref[...] = jnp.where(keep, s0, 0.0)


def _combine(pretrain_table, id_table, W_proj):
    w96 = jnp.concatenate(
        [W_proj.T, jnp.eye(EMBED_DIM, dtype=jnp.float32)], axis=0)
    wfull = jnp.kron(jnp.eye(_PACK, dtype=jnp.float32), w96)
    packed = pl.pallas_call(
        _combine_body,
        grid=(_GRID,),
        in_specs=[
            pl.BlockSpec((PRETRAIN_DIM, _BLK), lambda i: (0, i)),
            pl.BlockSpec((EMBED_DIM, _BLK), lambda i: (0, i)),
            pl.BlockSpec((_PACK * _GDIM, _OUTW), lambda i: (0, 0)),
        ],
        out_specs=pl.BlockSpec((_SUB, _OUTW), lambda i: (i, 0)),
        out_shape=jax.ShapeDtypeStruct((_DROWS, _OUTW), jnp.float32),
    )(pretrain_table.T, id_table.T, wfull)
    return packed.reshape(_VPAD, EMBED_DIM)


_IDXW = 128
_STREAMS = 4
_CHUNK = _IDXW * _STREAMS


_UNROLL = 16
_TPAD = _CHUNK + 8


def _make_gather(b, l):
    n_tokens = b * l
    info = plsc.get_sparse_core_info()
    nc, ns = info.num_cores, info.num_subcores
    nw = nc * ns
    b_per_w = n_tokens // nw
    n_outer = b_per_w // _CHUNK
    chunks_per_l = b // _CHUNK
    mesh = plsc.VectorSubcoreMesh(core_axis_name="c", subcore_axis_name="s")

    @functools.partial(
        pl.kernel,
        mesh=mesh,
        out_type=jax.ShapeDtypeStruct((l, EMBED_DIM, b), jnp.float32),
        scratch_types=[
            pltpu.VMEM((_STREAMS, _IDXW), jnp.int32),
            pltpu.VMEM((_STREAMS, _IDXW), jnp.int32),
            pltpu.VMEM((_CHUNK, EMBED_DIM), jnp.float32),
            pltpu.VMEM((_CHUNK, EMBED_DIM), jnp.float32),
            pltpu.VMEM((EMBED_DIM * _TPAD,), jnp.float32),
            pltpu.VMEM((EMBED_DIM * _TPAD,), jnp.float32),
            pltpu.SemaphoreType.DMA,
            pltpu.SemaphoreType.DMA,
            pltpu.SemaphoreType.DMA,
            pltpu.SemaphoreType.DMA,
            pltpu.SemaphoreType.DMA,
            pltpu.SemaphoreType.DMA,
        ],
        compiler_params=pltpu.CompilerParams(
            use_tc_tiling_on_sc=False, needs_layout_passes=False),
    )
    def gather_k(table_hbm, idx_hbm, out_hbm, idx0, idx1, rows0, rows1,
                 t0, t1, si0, si1, sg0, sg1, so0, so1):
        idx = (idx0, idx1)
        rows = (rows0, rows1)
        t = (t0, t1)
        si = (si0, si1)
        sg = (sg0, sg1)
        so = (so0, so1)
        wid = lax.axis_index("s") * nc + lax.axis_index("c")
        irow_base = wid * (b_per_w // _IDXW)
        lane = lax.iota(jnp.int32, 16)
        col0 = lane * _TPAD
        col1 = col0 + 16 * _TPAD
        n = n_outer

        def idx_slice(i):
            return idx_hbm.at[
                pl.ds(pl.multiple_of(irow_base + i * _STREAMS, _STREAMS),
                      _STREAMS)]

        def fire_idx(i, p):
            pltpu.async_copy(idx_slice(i), idx[p], si[p])

        def wait_idx(i, p):
            pltpu.make_async_copy(idx_slice(i), idx[p], si[p]).wait()

        def fire_gathers(p):
            for j in range(_STREAMS):
                pltpu.async_copy(
                    table_hbm.at[idx[p].at[j]],
                    rows[p].at[pl.ds(j * _IDXW, _IDXW)],
                    sg[p],
                )

        def wait_gathers(p):
            pltpu.make_async_copy(
                table_hbm.at[pl.ds(0, _CHUNK)], rows[p], sg[p]).wait()

        def drain_out(p):
            pltpu.make_async_copy(
                out_hbm.at[0, 0], t[p].at[pl.ds(0, EMBED_DIM * _CHUNK)],
                so[p]).wait()

        def chunk_pos(i):
            c = wid * n + i
            return c // chunks_per_l, (c % chunks_per_l) * _CHUNK

        def fire_out(i, p):
            li, b0 = chunk_pos(i)
            b0 = pl.multiple_of(b0, _CHUNK)
            for e in range(EMBED_DIM):
                pltpu.async_copy(
                    t[p].at[pl.ds(e * _TPAD, _CHUNK)],
                    out_hbm.at[li, e, pl.ds(b0, _CHUNK)],
                    so[p],
                )

        def transpose(p):
            rv, tv = rows[p], t[p]

            @plsc.parallel_loop(0, _CHUNK, unroll=_UNROLL)
            def trow(rr):
                x0 = rv[rr, pl.ds(0, 16)]
                x1 = rv[rr, pl.ds(16, 16)]
                plsc.store_scatter(tv, [col0 + rr], x0)
                plsc.store_scatter(tv, [col1 + rr], x1)

        fire_idx(0, 0)
        fire_idx(1, 1)
        wait_idx(0, 0)
        fire_gathers(0)

        def step(g, _):
            for p in range(2):
                i = 2 * g + p
                q = 1 - p

                with jax.named_scope("fg"):
                    @pl.when(i + 1 < n)
                    def _():
                        wait_idx(i + 1, q)
                        fire_gathers(q)

                with jax.named_scope("gw"):
                    wait_gathers(p)

                    @pl.when(i + 2 < n)
                    def _():
                        fire_idx(i + 2, p)

                    @pl.when(i >= 2)
                    def _():
                        drain_out(p)

                with jax.named_scope("tr"):
                    transpose(p)
                with jax.named_scope("fo"):
                    fire_out(i, p)
            return _

        lax.fori_loop(0, n // 2, step, None)
        drain_out(0)
        drain_out(1)

    return gather_k


def kernel(inputs, pretrain_table, id_table, W_proj):
    b, l = inputs.shape
    n_tokens = b * l
    table = _combine(pretrain_table, id_table, W_proj)
    v = inputs.T.astype(jnp.int32)
    blk = v // _SUB
    k = ((blk // _PACK) * _SUB + v % _SUB) * _PACK + blk % _PACK
    idx2d = k.reshape(n_tokens // _IDXW, _IDXW)
    out_t = _make_gather(b, l)(table, idx2d)
    return out_t.transpose(2, 0, 1)


# speedup vs baseline: 1.6899x; 1.0293x over previous
import functools

import jax
import jax.numpy as jnp
from jax import lax
from jax.experimental import pallas as pl
from jax.experimental.pallas import tpu as pltpu
from jax.experimental.pallas import tpu_sc as plsc

VOCAB = 1000000
PRETRAIN_DIM = 64
EMBED_DIM = 32
OOV_IDX = 999997


_SUB = 4096
_PACK = 4
_BLK = _SUB * _PACK
_GRID = -(-VOCAB // _BLK)
_DROWS = _GRID * _SUB
_VPAD = _DROWS * _PACK


_GDIM = PRETRAIN_DIM + EMBED_DIM
_OUTW = _PACK * EMBED_DIM


def _combine_body(pt_ref, it_ref, wf_ref, o_ref):
    i = pl.program_id(0)

    def slot_dots(j):
        proj = lax.dot_general(
            pt_ref[:, j * _SUB:(j + 1) * _SUB],
            wf_ref[j * _GDIM:j * _GDIM + PRETRAIN_DIM, :],
            dimension_numbers=(((0,), (0,)), ((), ())),
            preferred_element_type=jnp.float32,
        )
        idt = lax.dot_general(
            it_ref[:, j * _SUB:(j + 1) * _SUB],
            wf_ref[j * _GDIM + PRETRAIN_DIM:(j + 1) * _GDIM, :],
            dimension_numbers=(((0,), (0,)), ((), ())),
            preferred_element_type=jnp.float32,
        )
        return proj + idt

    @pl.when(i < _GRID - 1)
    def _():
        acc = None
        for j in range(_PACK):
            s = slot_dots(j)
            acc = s if acc is None else acc + s
        o_ref[...] = acc

    @pl.when(i == _GRID - 1)
    def _():
        s0 = slot_dots(0)
        col = lax.broadcasted_iota(jnp.int32, (_SUB, _OUTW), 1)
        v0 = i * _PACK * _SUB + lax.broadcasted_iota(
            jnp.int32, (_SUB, _OUTW), 0)
        keep = (col < EMBED_DIM) & (v0 <= OOV_IDX)
        o_ref[...] = jnp.where(keep, s0, 0.0)


def _combine(pretrain_table, id_table, W_proj):
    w96 = jnp.concatenate(
        [W_proj.T, jnp.eye(EMBED_DIM, dtype=jnp.float32)], axis=0)
    wfull = jnp.kron(jnp.eye(_PACK, dtype=jnp.float32), w96)
    packed = pl.pallas_call(
        _combine_body,
        grid=(_GRID,),
        in_specs=[
            pl.BlockSpec((PRETRAIN_DIM, _BLK), lambda i: (0, i)),
            pl.BlockSpec((EMBED_DIM, _BLK), lambda i: (0, i)),
            pl.BlockSpec((_PACK * _GDIM, _OUTW), lambda i: (0, 0)),
        ],
        out_specs=pl.BlockSpec((_SUB, _OUTW), lambda i: (i, 0)),
        out_shape=jax.ShapeDtypeStruct((_DROWS, _OUTW), jnp.float32),
    )(pretrain_table.T, id_table.T, wfull)
    return packed.reshape(_VPAD, EMBED_DIM)


_IDXW = 128
_STREAMS = 4
_CHUNK = _IDXW * _STREAMS


_UNROLL = 16
_TPAD = _CHUNK + 8


def _make_gather(b, l):
    n_tokens = b * l
    info = plsc.get_sparse_core_info()
    nc, ns = info.num_cores, info.num_subcores
    nw = nc * ns
    b_per_w = n_tokens // nw
    n_outer = b_per_w // _CHUNK
    chunks_per_l = b // _CHUNK
    mesh = plsc.VectorSubcoreMesh(core_axis_name="c", subcore_axis_name="s")

    @functools.partial(
        pl.kernel,
        mesh=mesh,
        out_type=jax.ShapeDtypeStruct((l, EMBED_DIM, b), jnp.float32),
        scratch_types=[
            pltpu.VMEM((_STREAMS, _IDXW), jnp.int32),
            pltpu.VMEM((_STREAMS, _IDXW), jnp.int32),
            pltpu.VMEM((_CHUNK, EMBED_DIM), jnp.float32),
            pltpu.VMEM((_CHUNK, EMBED_DIM), jnp.float32),
            pltpu.VMEM((EMBED_DIM * _TPAD,), jnp.float32),
            pltpu.VMEM((EMBED_DIM * _TPAD,), jnp.float32),
            pltpu.SemaphoreType.DMA,
            pltpu.SemaphoreType.DMA,
            pltpu.SemaphoreType.DMA,
            pltpu.SemaphoreType.DMA,
            pltpu.SemaphoreType.DMA,
            pltpu.SemaphoreType.DMA,
        ],
        compiler_params=pltpu.CompilerParams(
            use_tc_tiling_on_sc=False, needs_layout_passes=False),
    )
    def gather_k(table_hbm, idx_hbm, out_hbm, idx0, idx1, rows0, rows1,
                 t0, t1, si0, si1, sg0, sg1, so0, so1):
        idx = (idx0, idx1)
        rows = (rows0, rows1)
        t = (t0, t1)
        si = (si0, si1)
        sg = (sg0, sg1)
        so = (so0, so1)
        wid = lax.axis_index("s") * nc + lax.axis_index("c")
        irow_base = wid * (b_per_w // _IDXW)
        lane = lax.iota(jnp.int32, 16)
        col0 = lane * _TPAD
        col1 = col0 + 16 * _TPAD
        n = n_outer

        def idx_slice(i):
            return idx_hbm.at[
                pl.ds(pl.multiple_of(irow_base + i * _STREAMS, _STREAMS),
                      _STREAMS)]

        def fire_idx(i, p):
            pltpu.async_copy(idx_slice(i), idx[p], si[p])

        def wait_idx(i, p):
            pltpu.make_async_copy(idx_slice(i), idx[p], si[p]).wait()

        def fire_gathers(p):
            for j in range(_STREAMS):
                pltpu.async_copy(
                    table_hbm.at[idx[p].at[j]],
                    rows[p].at[pl.ds(j * _IDXW, _IDXW)],
                    sg[p],
                )

        def wait_gathers(p):
            pltpu.make_async_copy(
                table_hbm.at[pl.ds(0, _CHUNK)], rows[p], sg[p]).wait()

        def drain_out(p):
            pltpu.make_async_copy(
                out_hbm.at[0, 0], t[p].at[pl.ds(0, EMBED_DIM * _CHUNK)],
                so[p]).wait()

        def chunk_pos(i):
            c = wid * n + i
            return c // chunks_per_l, (c % chunks_per_l) * _CHUNK

        def fire_out(i, p):
            li, b0 = chunk_pos(i)
            b0 = pl.multiple_of(b0, _CHUNK)
            for e in range(EMBED_DIM):
                pltpu.async_copy(
                    t[p].at[pl.ds(e * _TPAD, _CHUNK)],
                    out_hbm.at[li, e, pl.ds(b0, _CHUNK)],
                    so[p],
                )

        def transpose(p):
            rv, tv = rows[p], t[p]

            @plsc.parallel_loop(0, _CHUNK, unroll=_UNROLL)
            def trow(rr):
                x0 = rv[rr, pl.ds(0, 16)]
                x1 = rv[rr, pl.ds(16, 16)]
                plsc.store_scatter(tv, [col0 + rr], x0)
                plsc.store_scatter(tv, [col1 + rr], x1)

        fire_idx(0, 0)
        fire_idx(1, 1)
        wait_idx(0, 0)
        fire_gathers(0)

        def step(g, _):
            for p in range(2):
                i = 2 * g + p
                q = 1 - p

                with jax.named_scope("fg"):
                    @pl.when(i + 1 < n)
                    def _():
                        wait_idx(i + 1, q)
                        fire_gathers(q)

                with jax.named_scope("gw"):
                    wait_gathers(p)

                    @pl.when(i + 2 < n)
                    def _():
                        fire_idx(i + 2, p)

                    @pl.when(i >= 2)
                    def _():
                        drain_out(p)

                with jax.named_scope("tr"):
                    transpose(p)
                with jax.named_scope("fo"):
                    fire_out(i, p)
            return _

        lax.fori_loop(0, n // 2, step, None)
        drain_out(0)
        drain_out(1)

    return gather_k


def kernel(inputs, pretrain_table, id_table, W_proj):
    b, l = inputs.shape
    n_tokens = b * l
    table = _combine(pretrain_table, id_table, W_proj)
    v = inputs.T.astype(jnp.int32)
    blk = v // _SUB
    k = ((blk // _PACK) * _SUB + v % _SUB) * _PACK + blk % _PACK
    idx2d = k.reshape(n_tokens // _IDXW, _IDXW)
    out_t = _make_gather(b, l)(table, idx2d)
    return out_t.transpose(2, 0, 1)


# speedup vs baseline: 2.0000x; 1.1835x over previous
import functools

import jax
import jax.numpy as jnp
from jax import lax
from jax.experimental import pallas as pl
from jax.experimental.pallas import tpu as pltpu
from jax.experimental.pallas import tpu_sc as plsc

VOCAB = 1000000
PRETRAIN_DIM = 64
EMBED_DIM = 32
OOV_IDX = 999997


_SUB = 4096
_PACK = 4
_BLK = _SUB * _PACK
_GRID = -(-VOCAB // _BLK)
_DROWS = _GRID * _SUB
_VPAD = _DROWS * _PACK


_GDIM = PRETRAIN_DIM + EMBED_DIM
_OUTW = _PACK * EMBED_DIM


def _combine_body(pt_ref, it_ref, wf_ref, o_ref):
    i = pl.program_id(0)
    ptb = pt_ref[...].astype(jnp.bfloat16)
    itb = it_ref[...].astype(jnp.bfloat16)

    def slot_dots(j):
        proj = lax.dot_general(
            ptb[:, j * _SUB:(j + 1) * _SUB],
            wf_ref[j * _GDIM:j * _GDIM + PRETRAIN_DIM, :],
            dimension_numbers=(((0,), (0,)), ((), ())),
            preferred_element_type=jnp.float32,
        )
        idt = lax.dot_general(
            itb[:, j * _SUB:(j + 1) * _SUB],
            wf_ref[j * _GDIM + PRETRAIN_DIM:(j + 1) * _GDIM, :],
            dimension_numbers=(((0,), (0,)), ((), ())),
            preferred_element_type=jnp.float32,
        )
        return proj + idt

    @pl.when(i < _GRID - 1)
    def _():
        acc = None
        for j in range(_PACK):
            s = slot_dots(j)
            acc = s if acc is None else acc + s
        o_ref[...] = acc

    @pl.when(i == _GRID - 1)
    def _():
        s0 = slot_dots(0)
        col = lax.broadcasted_iota(jnp.int32, (_SUB, _OUTW), 1)
        v0 = i * _PACK * _SUB + lax.broadcasted_iota(
            jnp.int32, (_SUB, _OUTW), 0)
        keep = (col < EMBED_DIM) & (v0 <= OOV_IDX)
        o_ref[...] = jnp.where(keep, s0, 0.0)


def _combine(pretrain_table, id_table, W_proj):
    w96 = jnp.concatenate(
        [W_proj.T, jnp.eye(EMBED_DIM, dtype=jnp.float32)], axis=0)
    wfull = jnp.kron(
        jnp.eye(_PACK, dtype=jnp.float32), w96).astype(jnp.bfloat16)
    packed = pl.pallas_call(
        _combine_body,
        grid=(_GRID,),
        in_specs=[
            pl.BlockSpec((PRETRAIN_DIM, _BLK), lambda i: (0, i)),
            pl.BlockSpec((EMBED_DIM, _BLK), lambda i: (0, i)),
            pl.BlockSpec((_PACK * _GDIM, _OUTW), lambda i: (0, 0)),
        ],
        out_specs=pl.BlockSpec((_SUB, _OUTW), lambda i: (i, 0)),
        out_shape=jax.ShapeDtypeStruct((_DROWS, _OUTW), jnp.float32),
    )(pretrain_table.T, id_table.T, wfull)
    return packed.reshape(_VPAD, EMBED_DIM)


_IDXW = 128
_STREAMS = 4
_CHUNK = _IDXW * _STREAMS


_UNROLL = 16
_TPAD = _CHUNK + 8


def _make_gather(b, l):
    n_tokens = b * l
    info = plsc.get_sparse_core_info()
    nc, ns = info.num_cores, info.num_subcores
    nw = nc * ns
    b_per_w = n_tokens // nw
    n_outer = b_per_w // _CHUNK
    chunks_per_l = b // _CHUNK
    mesh = plsc.VectorSubcoreMesh(core_axis_name="c", subcore_axis_name="s")

    @functools.partial(
        pl.kernel,
        mesh=mesh,
        out_type=jax.ShapeDtypeStruct((l, EMBED_DIM, b), jnp.float32),
        scratch_types=[
            pltpu.VMEM((_STREAMS, _IDXW), jnp.int32),
            pltpu.VMEM((_STREAMS, _IDXW), jnp.int32),
            pltpu.VMEM((_CHUNK, EMBED_DIM), jnp.float32),
            pltpu.VMEM((_CHUNK, EMBED_DIM), jnp.float32),
            pltpu.VMEM((EMBED_DIM * _TPAD,), jnp.float32),
            pltpu.VMEM((EMBED_DIM * _TPAD,), jnp.float32),
            pltpu.SemaphoreType.DMA,
            pltpu.SemaphoreType.DMA,
            pltpu.SemaphoreType.DMA,
            pltpu.SemaphoreType.DMA,
            pltpu.SemaphoreType.DMA,
            pltpu.SemaphoreType.DMA,
        ],
        compiler_params=pltpu.CompilerParams(
            use_tc_tiling_on_sc=False, needs_layout_passes=False),
    )
    def gather_k(table_hbm, idx_hbm, out_hbm, idx0, idx1, rows0, rows1,
                 t0, t1, si0, si1, sg0, sg1, so0, so1):
        idx = (idx0, idx1)
        rows = (rows0, rows1)
        t = (t0, t1)
        si = (si0, si1)
        sg = (sg0, sg1)
        so = (so0, so1)
        wid = lax.axis_index("s") * nc + lax.axis_index("c")
        irow_base = wid * (b_per_w // _IDXW)
        lane = lax.iota(jnp.int32, 16)
        col0 = lane * _TPAD
        col1 = col0 + 16 * _TPAD
        n = n_outer

        def idx_slice(i):
            return idx_hbm.at[
                pl.ds(pl.multiple_of(irow_base + i * _STREAMS, _STREAMS),
                      _STREAMS)]

        def fire_idx(i, p):
            pltpu.async_copy(idx_slice(i), idx[p], si[p])

        def wait_idx(i, p):
            pltpu.make_async_copy(idx_slice(i), idx[p], si[p]).wait()

        def fire_gathers(p):
            for j in range(_STREAMS):
                pltpu.async_copy(
                    table_hbm.at[idx[p].at[j]],
                    rows[p].at[pl.ds(j * _IDXW, _IDXW)],
                    sg[p],
                )

        def wait_gathers(p):
            pltpu.make_async_copy(
                table_hbm.at[pl.ds(0, _CHUNK)], rows[p], sg[p]).wait()

        def drain_out(p):
            pltpu.make_async_copy(
                out_hbm.at[0, 0], t[p].at[pl.ds(0, EMBED_DIM * _CHUNK)],
                so[p]).wait()

        def chunk_pos(i):
            c = wid * n + i
            return c // chunks_per_l, (c % chunks_per_l) * _CHUNK

        def fire_out(i, p):
            li, b0 = chunk_pos(i)
            b0 = pl.multiple_of(b0, _CHUNK)
            for e in range(EMBED_DIM):
                pltpu.async_copy(
                    t[p].at[pl.ds(e * _TPAD, _CHUNK)],
                    out_hbm.at[li, e, pl.ds(b0, _CHUNK)],
                    so[p],
                )

        def transpose(p):
            rv, tv = rows[p], t[p]

            @plsc.parallel_loop(0, _CHUNK, unroll=_UNROLL)
            def trow(rr):
                x0 = rv[rr, pl.ds(0, 16)]
                x1 = rv[rr, pl.ds(16, 16)]
                plsc.store_scatter(tv, [col0 + rr], x0)
                plsc.store_scatter(tv, [col1 + rr], x1)

        fire_idx(0, 0)
        fire_idx(1, 1)
        wait_idx(0, 0)
        fire_gathers(0)

        def step(g, _):
            for p in range(2):
                i = 2 * g + p
                q = 1 - p

                with jax.named_scope("fg"):
                    @pl.when(i + 1 < n)
                    def _():
                        wait_idx(i + 1, q)
                        fire_gathers(q)

                with jax.named_scope("gw"):
                    wait_gathers(p)

                    @pl.when(i + 2 < n)
                    def _():
                        fire_idx(i + 2, p)

                    @pl.when(i >= 2)
                    def _():
                        drain_out(p)

                with jax.named_scope("tr"):
                    transpose(p)
                with jax.named_scope("fo"):
                    fire_out(i, p)
            return _

        lax.fori_loop(0, n // 2, step, None)
        drain_out(0)
        drain_out(1)

    return gather_k


def kernel(inputs, pretrain_table, id_table, W_proj):
    b, l = inputs.shape
    n_tokens = b * l
    table = _combine(pretrain_table, id_table, W_proj)
    v = inputs.T.astype(jnp.int32)
    blk = v // _SUB
    k = ((blk // _PACK) * _SUB + v % _SUB) * _PACK + blk % _PACK
    idx2d = k.reshape(n_tokens // _IDXW, _IDXW)
    out_t = _make_gather(b, l)(table, idx2d)
    return out_t.transpose(2, 0, 1)
